# baseline (speedup 1.0000x reference)
import jax
import jax.numpy as jnp
from jax import lax
from jax.experimental import pallas as pl
from jax.experimental.pallas import tpu as pltpu
from jax.experimental.pallas import tpu_sc as plsc

VOCAB = 1000000
EMBED = 64
B = 4096
L = 200
CH0 = 128
CH1 = L - CH0

_info = plsc.get_sparse_core_info()
NC, NS, NL = _info.num_cores, _info.num_subcores, _info.num_lanes
NW = NC * NS
RPW = B // NW
HALF = RPW // 2
NACC = EMBED // NL

PBLK = 4000


def _tcpack_body(x_ref, o_ref):
    x3 = x_ref[...].reshape(PBLK // 2, 2, EMBED)
    o_ref[...] = jnp.concatenate([x3[:, 0, :], x3[:, 1, :]], axis=-1)


def _tcpack(table):
    return pl.pallas_call(
        _tcpack_body,
        grid=(VOCAB // PBLK,),
        in_specs=[pl.BlockSpec((PBLK, EMBED), lambda i: (i, 0))],
        out_specs=pl.BlockSpec((PBLK // 2, 2 * EMBED), lambda i: (i, 0)),
        out_shape=jax.ShapeDtypeStruct((VOCAB // 2, 2 * EMBED), jnp.float32),
    )(table)


def _pool_body(idx_hbm, table_hbm, out_hbm, idx_v, rows0, rows1, pooled_v,
               sem0, sem1):
    wid = lax.axis_index("s") * NC + lax.axis_index("c")
    base = wid * RPW
    pltpu.sync_copy(idx_hbm.at[pl.ds(base, RPW)], idx_v)

    def issue(b, rows, sem):
        pltpu.async_copy(table_hbm.at[idx_v.at[b, pl.ds(0, CH0)]],
                         rows.at[pl.ds(0, CH0)], sem)
        pltpu.async_copy(table_hbm.at[idx_v.at[b, pl.ds(CH0, CH1)]],
                         rows.at[pl.ds(CH0, CH1)], sem)

    def drain(rows, sem):
        pltpu.make_async_copy(table_hbm.at[pl.ds(0, L)], rows, sem).wait()

    def reduce_into(rows, b):
        def rbody(j, accs):
            return tuple(accs[k] + rows[j, pl.ds(NL * k, NL)]
                         for k in range(NACC))
        z = jnp.zeros((NL,), jnp.float32)
        accs = lax.fori_loop(0, L, rbody, (z,) * NACC, unroll=8)
        for k in range(NACC):
            pooled_v[b, pl.ds(NL * k, NL)] = accs[k]

    issue(0, rows0, sem0)

    def body(i, carry):
        b = 2 * i
        issue(b + 1, rows1, sem1)
        drain(rows0, sem0)
        reduce_into(rows0, b)

        @pl.when(i < HALF - 1)
        def _():
            issue(b + 2, rows0, sem0)

        drain(rows1, sem1)
        reduce_into(rows1, b + 1)
        return carry

    lax.fori_loop(0, HALF, body, 0)
    pltpu.sync_copy(pooled_v, out_hbm.at[pl.ds(base, RPW)])


def _pool(inputs, table_lin):
    mesh = plsc.VectorSubcoreMesh(core_axis_name="c", subcore_axis_name="s")
    f = pl.kernel(
        _pool_body,
        mesh=mesh,
        compiler_params=pltpu.CompilerParams(use_tc_tiling_on_sc=False),
        out_type=jax.ShapeDtypeStruct((B, EMBED), jnp.float32),
        scratch_types=[
            pltpu.VMEM((RPW, L), jnp.int32),
            pltpu.VMEM((L, EMBED), jnp.float32),
            pltpu.VMEM((L, EMBED), jnp.float32),
            pltpu.VMEM((RPW, EMBED), jnp.float32),
            pltpu.SemaphoreType.DMA,
            pltpu.SemaphoreType.DMA,
        ],
    )
    return f(inputs, table_lin)


def _mlp_body(x_ref, w1_ref, b1_ref, w2_ref, b2_ref, w3_ref, b3_ref, o_ref):
    h = jnp.tanh(jnp.dot(x_ref[...], w1_ref[...],
                         preferred_element_type=jnp.float32) + b1_ref[...])
    h = jnp.tanh(jnp.dot(h, w2_ref[...],
                         preferred_element_type=jnp.float32) + b2_ref[...])
    o_ref[...] = jnp.dot(h, w3_ref[...],
                         preferred_element_type=jnp.float32) + b3_ref[...]


def _mlp(x, W1, b1, W2, b2, W3, b3):
    ncls = W3.shape[1]
    pad = 128 - ncls
    W3p = jnp.pad(W3, ((0, 0), (0, pad)))
    b3p = jnp.pad(b3, (0, pad))
    out = pl.pallas_call(
        _mlp_body,
        out_shape=jax.ShapeDtypeStruct((B, 128), jnp.float32),
    )(x, W1, b1.reshape(1, -1), W2, b2.reshape(1, -1),
      W3p, b3p.reshape(1, -1))
    return out[:, :ncls]


def kernel(inputs, table, W1, b1, W2, b2, W3, b3):
    packed = _tcpack(table)
    table_lin = packed.reshape(VOCAB, EMBED)
    pooled = _pool(inputs, table_lin)
    return _mlp(pooled, W1, b1, W2, b2, W3, b3)


# speedup vs baseline: 1.2568x; 1.2568x over previous
import jax
import jax.numpy as jnp
from jax import lax
from jax.experimental import pallas as pl
from jax.experimental.pallas import tpu as pltpu
from jax.experimental.pallas import tpu_sc as plsc

EMBED = 64
B = 4096
L = 200
CH0 = 128
CH1 = L - CH0

_info = plsc.get_sparse_core_info()
NC, NS, NL = _info.num_cores, _info.num_subcores, _info.num_lanes
NW = NC * NS
RPW = B // NW
HALF = RPW // 2
NACC = EMBED // NL


def _pool_body(idx_hbm, table_hbm, out_hbm, idx_v, rows0, rows1, pooled_v,
               sem0, sem1):
    wid = lax.axis_index("s") * NC + lax.axis_index("c")
    base = wid * RPW
    pltpu.sync_copy(idx_hbm.at[pl.ds(base, RPW)], idx_v)

    def issue(b, rows, sem):
        pltpu.async_copy(table_hbm.at[idx_v.at[b, pl.ds(0, CH0)]],
                         rows.at[pl.ds(0, CH0)], sem)
        pltpu.async_copy(table_hbm.at[idx_v.at[b, pl.ds(CH0, CH1)]],
                         rows.at[pl.ds(CH0, CH1)], sem)

    def drain(rows, sem):
        pltpu.make_async_copy(table_hbm.at[pl.ds(0, L)], rows, sem).wait()

    def reduce_into(rows, b):
        def rbody(j, accs):
            return tuple(accs[k] + rows[j, pl.ds(NL * k, NL)]
                         for k in range(NACC))
        z = jnp.zeros((NL,), jnp.float32)
        accs = lax.fori_loop(0, L, rbody, (z,) * NACC, unroll=8)
        for k in range(NACC):
            pooled_v[b, pl.ds(NL * k, NL)] = accs[k]

    issue(0, rows0, sem0)

    def body(i, carry):
        b = 2 * i
        issue(b + 1, rows1, sem1)
        drain(rows0, sem0)
        reduce_into(rows0, b)

        @pl.when(i < HALF - 1)
        def _():
            issue(b + 2, rows0, sem0)

        drain(rows1, sem1)
        reduce_into(rows1, b + 1)
        return carry

    lax.fori_loop(0, HALF, body, 0)
    pltpu.sync_copy(pooled_v, out_hbm.at[pl.ds(base, RPW)])


def _pool(inputs, table):
    mesh = plsc.VectorSubcoreMesh(core_axis_name="c", subcore_axis_name="s")
    f = pl.kernel(
        _pool_body,
        mesh=mesh,
        compiler_params=pltpu.CompilerParams(use_tc_tiling_on_sc=False,
                                             needs_layout_passes=True),
        out_type=jax.ShapeDtypeStruct((B, EMBED), jnp.float32),
        scratch_types=[
            pltpu.VMEM((RPW, L), jnp.int32),
            pltpu.VMEM((L, EMBED), jnp.float32),
            pltpu.VMEM((L, EMBED), jnp.float32),
            pltpu.VMEM((RPW, EMBED), jnp.float32),
            pltpu.SemaphoreType.DMA,
            pltpu.SemaphoreType.DMA,
        ],
    )
    return f(inputs, table)


def _mlp_body(x_ref, w1_ref, b1_ref, w2_ref, b2_ref, w3_ref, b3_ref, o_ref):
    h = jnp.tanh(jnp.dot(x_ref[...], w1_ref[...],
                         preferred_element_type=jnp.float32) + b1_ref[...])
    h = jnp.tanh(jnp.dot(h, w2_ref[...],
                         preferred_element_type=jnp.float32) + b2_ref[...])
    o_ref[...] = jnp.dot(h, w3_ref[...],
                         preferred_element_type=jnp.float32) + b3_ref[...]


def _mlp(x, W1, b1, W2, b2, W3, b3):
    ncls = W3.shape[1]
    pad = 128 - ncls
    W3p = jnp.pad(W3, ((0, 0), (0, pad)))
    b3p = jnp.pad(b3, (0, pad))
    out = pl.pallas_call(
        _mlp_body,
        out_shape=jax.ShapeDtypeStruct((B, 128), jnp.float32),
    )(x, W1, b1.reshape(1, -1), W2, b2.reshape(1, -1),
      W3p, b3p.reshape(1, -1))
    return out[:, :ncls]


def kernel(inputs, table, W1, b1, W2, b2, W3, b3):
    pooled = _pool(inputs, table)
    return _mlp(pooled, W1, b1, W2, b2, W3, b3)
